# baseline (speedup 1.0000x reference)
import functools

import jax
import jax.numpy as jnp
from jax import lax
from jax.experimental import pallas as pl
from jax.experimental.pallas import tpu as pltpu
from jax.experimental.pallas import tpu_sc as plsc

N_NODES = 10000
D = 256
HALF = 128
N_PAD = 10240
E_PAD = 163840
NC = 2
NS = 16
NTILES = NC * NS
BLK = 128
NBLK = E_PAD // (NTILES * BLK)
RPT = N_PAD // NS
RB = 640
NBUF = 2


def _sc_degree(dst_blocks, zeros128, ones128):
    mesh = plsc.VectorSubcoreMesh(core_axis_name="c", subcore_axis_name="s")

    @functools.partial(
        pl.kernel,
        mesh=mesh,
        out_type=jax.ShapeDtypeStruct((NC, N_PAD, HALF), jnp.float32),
        scratch_types=[
            pltpu.VMEM((NBLK, BLK), jnp.int32),
            pltpu.VMEM((BLK, HALF), jnp.float32),
            pltpu.VMEM_SHARED((N_PAD, HALF), jnp.float32),
        ],
    )
    def k(dst_r, z_r, o_r, out_r, didx, ones_v, acc):
        cid = lax.axis_index("c")
        sid = lax.axis_index("s")
        wid = cid * NS + sid
        rows = pl.ds(sid * RPT, RPT)
        pltpu.sync_copy(z_r.at[rows], acc.at[rows])
        pltpu.sync_copy(o_r, ones_v)
        pltpu.sync_copy(dst_r.at[wid], didx)
        plsc.subcore_barrier()

        def body(j, carry):
            pltpu.sync_copy(ones_v, acc.at[didx.at[j]], add=True)
            return carry

        lax.fori_loop(0, NBLK, body, 0)
        plsc.subcore_barrier()
        pltpu.sync_copy(acc.at[rows], out_r.at[cid].at[rows])

    return k(dst_blocks, zeros128, ones128)


def _sc_propagate(t0, t1, src_blocks, dst_blocks, zeros128):
    mesh = plsc.VectorSubcoreMesh(core_axis_name="c", subcore_axis_name="s")

    @functools.partial(
        pl.kernel,
        mesh=mesh,
        out_type=jax.ShapeDtypeStruct((NC, 2, N_PAD, HALF), jnp.float32),
        scratch_types=[
            pltpu.VMEM((NBLK, BLK), jnp.int32),
            pltpu.VMEM((NBLK, BLK), jnp.int32),
            pltpu.VMEM((NBUF, BLK, HALF), jnp.float32),
            pltpu.VMEM_SHARED((N_PAD, HALF), jnp.float32),
        ]
        + [pltpu.SemaphoreType.DMA] * NBUF,
    )
    def k(t0_r, t1_r, src_r, dst_r, z_r, out_r, sidx, didx, rows_v, acc, *sems):
        cid = lax.axis_index("c")
        sid = lax.axis_index("s")
        wid = cid * NS + sid
        rows = pl.ds(sid * RPT, RPT)
        pltpu.sync_copy(src_r.at[wid], sidx)
        pltpu.sync_copy(dst_r.at[wid], didx)

        def drain(b):
            pltpu.make_async_copy(
                z_r.at[pl.ds(0, BLK)], rows_v.at[b], sems[b]
            ).wait()

        for half, t_r in ((0, t0_r), (1, t1_r)):
            pltpu.sync_copy(z_r.at[rows], acc.at[rows])
            plsc.subcore_barrier()
            for b in range(NBUF):
                pltpu.async_copy(t_r.at[sidx.at[b]], rows_v.at[b], sems[b])

            def round_(g, carry):
                base = g * NBUF
                for b in range(NBUF):
                    drain(b)
                    pltpu.async_copy(
                        rows_v.at[b], acc.at[didx.at[base + b]], sems[b],
                        add=True,
                    )
                for b in range(NBUF):
                    nj = base + NBUF + b

                    @pl.when(nj < NBLK)
                    def _(b=b, nj=nj):
                        drain(b)
                        pltpu.async_copy(
                            t_r.at[sidx.at[nj]], rows_v.at[b], sems[b]
                        )

                return carry

            lax.fori_loop(0, NBLK // NBUF, round_, 0)
            for b in range(NBUF):
                drain(b)
            plsc.subcore_barrier()
            pltpu.sync_copy(acc.at[rows], out_r.at[cid, half].at[rows])

    return k(t0, t1, src_blocks, dst_blocks, zeros128)


def _dinv(d0_r, d1_r):
    deg = d0_r[:, :1] + d1_r[:, :1] + 1.0
    return lax.rsqrt(deg)


def _tc_stage1(x, W1, d0, d1):
    def body(x_r, w_r, d0_r, d1_r, h_r, t0_r, t1_r):
        dinv = _dinv(d0_r, d1_r)
        h = jnp.dot(x_r[...], w_r[...], preferred_element_type=jnp.float32)
        h_r[...] = h
        hs = h * dinv
        t0_r[...] = hs[:, :HALF]
        t1_r[...] = hs[:, HALF:]

    return pl.pallas_call(
        body,
        grid=(N_PAD // RB,),
        in_specs=[
            pl.BlockSpec((RB, D), lambda i: (i, 0)),
            pl.BlockSpec((D, D), lambda i: (0, 0)),
            pl.BlockSpec((RB, 16), lambda i: (i, 0)),
            pl.BlockSpec((RB, 16), lambda i: (i, 0)),
        ],
        out_specs=[
            pl.BlockSpec((RB, D), lambda i: (i, 0)),
            pl.BlockSpec((RB, HALF), lambda i: (i, 0)),
            pl.BlockSpec((RB, HALF), lambda i: (i, 0)),
        ],
        out_shape=[
            jax.ShapeDtypeStruct((N_PAD, D), jnp.float32),
            jax.ShapeDtypeStruct((N_PAD, HALF), jnp.float32),
            jax.ShapeDtypeStruct((N_PAD, HALF), jnp.float32),
        ],
    )(x, W1, d0, d1)


def _tc_stage2(a0, a1, h1, W2, b1, d0, d1):
    def body(a0_r, a1_r, h1_r, w_r, b_r, d0_r, d1_r, h2_r, u0_r, u1_r):
        dinv = _dinv(d0_r, d1_r)
        z = (a0_r[...] + a1_r[...]) * dinv + (dinv * dinv) * h1_r[...] + b_r[...]
        z = jnp.maximum(z, 0.0)
        h2 = jnp.dot(z, w_r[...], preferred_element_type=jnp.float32)
        h2_r[...] = h2
        hs = h2 * dinv
        u0_r[...] = hs[:, :HALF]
        u1_r[...] = hs[:, HALF:]

    return pl.pallas_call(
        body,
        grid=(N_PAD // RB,),
        in_specs=[
            pl.BlockSpec((RB, D), lambda i: (i, 0)),
            pl.BlockSpec((RB, D), lambda i: (i, 0)),
            pl.BlockSpec((RB, D), lambda i: (i, 0)),
            pl.BlockSpec((D, D), lambda i: (0, 0)),
            pl.BlockSpec((1, D), lambda i: (0, 0)),
            pl.BlockSpec((RB, 16), lambda i: (i, 0)),
            pl.BlockSpec((RB, 16), lambda i: (i, 0)),
        ],
        out_specs=[
            pl.BlockSpec((RB, D), lambda i: (i, 0)),
            pl.BlockSpec((RB, HALF), lambda i: (i, 0)),
            pl.BlockSpec((RB, HALF), lambda i: (i, 0)),
        ],
        out_shape=[
            jax.ShapeDtypeStruct((N_PAD, D), jnp.float32),
            jax.ShapeDtypeStruct((N_PAD, HALF), jnp.float32),
            jax.ShapeDtypeStruct((N_PAD, HALF), jnp.float32),
        ],
    )(a0, a1, h1, W2, b1, d0, d1)


def _tc_stage3(c0, c1, h2, b2, d0, d1):
    def body(c0_r, c1_r, h2_r, b_r, d0_r, d1_r, o_r):
        dinv = _dinv(d0_r, d1_r)
        o_r[...] = (
            (c0_r[...] + c1_r[...]) * dinv
            + (dinv * dinv) * h2_r[...]
            + b_r[...]
        )

    return pl.pallas_call(
        body,
        grid=(N_PAD // RB,),
        in_specs=[
            pl.BlockSpec((RB, D), lambda i: (i, 0)),
            pl.BlockSpec((RB, D), lambda i: (i, 0)),
            pl.BlockSpec((RB, D), lambda i: (i, 0)),
            pl.BlockSpec((1, D), lambda i: (0, 0)),
            pl.BlockSpec((RB, 16), lambda i: (i, 0)),
            pl.BlockSpec((RB, 16), lambda i: (i, 0)),
        ],
        out_specs=pl.BlockSpec((RB, D), lambda i: (i, 0)),
        out_shape=jax.ShapeDtypeStruct((N_PAD, D), jnp.float32),
    )(c0, c1, h2, b2, d0, d1)


def kernel(x, edge_index, W1, b1, W2, b2):
    src = edge_index[0].astype(jnp.int32)
    dst = edge_index[1].astype(jnp.int32)
    e = src.shape[0]
    padfill = jnp.full((E_PAD - e,), N_NODES, jnp.int32)
    srcb = jnp.concatenate([src, padfill]).reshape(NTILES, NBLK, BLK)
    dstb = jnp.concatenate([dst, padfill]).reshape(NTILES, NBLK, BLK)
    xp = jnp.zeros((N_PAD, D), jnp.float32).at[:N_NODES].set(x)
    z128 = jnp.zeros((N_PAD, HALF), jnp.float32)
    o128 = jnp.ones((BLK, HALF), jnp.float32)

    degp = _sc_degree(dstb, z128, o128)
    d0, d1 = degp[0, :, :16], degp[1, :, :16]

    h1, t0, t1 = _tc_stage1(xp, W1, d0, d1)
    p = _sc_propagate(t0, t1, srcb, dstb, z128)
    a0 = jnp.concatenate([p[0, 0], p[0, 1]], axis=1)
    a1 = jnp.concatenate([p[1, 0], p[1, 1]], axis=1)

    h2, u0, u1 = _tc_stage2(a0, a1, h1, W2, b1.reshape(1, D), d0, d1)
    q = _sc_propagate(u0, u1, srcb, dstb, z128)
    c0 = jnp.concatenate([q[0, 0], q[0, 1]], axis=1)
    c1 = jnp.concatenate([q[1, 0], q[1, 1]], axis=1)

    out = _tc_stage3(c0, c1, h2, b2.reshape(1, D), d0, d1)
    return out[:N_NODES]


# speedup vs baseline: 1.4916x; 1.4916x over previous
import functools

import jax
import jax.numpy as jnp
from jax import lax
from jax.experimental import pallas as pl
from jax.experimental.pallas import tpu as pltpu
from jax.experimental.pallas import tpu_sc as plsc

N_NODES = 10000
D = 256
HALF = 128
N_PAD = 10240
E_PAD = 163840
NC = 2
NS = 16
BLK = 128
NBLKT = E_PAD // (NS * BLK)
NCHUNK = 2
NBLK = NBLKT // NCHUNK
RPT = N_PAD // NS
CH = 64
RB = 640
NBUF = 2


def _sc_degree(dst_blocks, zeros128, ones128):
    mesh = plsc.VectorSubcoreMesh(core_axis_name="c", subcore_axis_name="s")

    @functools.partial(
        pl.kernel,
        mesh=mesh,
        out_type=jax.ShapeDtypeStruct((NC, N_PAD, HALF), jnp.float32),
        scratch_types=[
            pltpu.VMEM((NBLK, BLK), jnp.int32),
            pltpu.VMEM((BLK, HALF), jnp.float32),
            pltpu.VMEM_SHARED((N_PAD, HALF), jnp.float32),
        ],
    )
    def k(dst_r, z_r, o_r, out_r, didx, ones_v, acc):
        cid = lax.axis_index("c")
        sid = lax.axis_index("s")
        wid = cid * NS + sid
        rows = pl.ds(sid * RPT, RPT)
        pltpu.sync_copy(z_r.at[rows], acc.at[rows])
        pltpu.sync_copy(o_r, ones_v)
        pltpu.sync_copy(dst_r.at[wid], didx)
        plsc.subcore_barrier()

        def body(j, carry):
            pltpu.sync_copy(ones_v, acc.at[didx.at[j]], add=True)
            return carry

        lax.fori_loop(0, NBLK, body, 0)
        plsc.subcore_barrier()
        pltpu.sync_copy(acc.at[rows], out_r.at[cid].at[rows])

    return k(dst_blocks, zeros128, ones128)


def _sc_propagate(t, src_blocks, dst_blocks, zeros128, dinv, b, fuse_out):
    mesh = plsc.VectorSubcoreMesh(core_axis_name="c", subcore_axis_name="s")

    @functools.partial(
        pl.kernel,
        mesh=mesh,
        out_type=jax.ShapeDtypeStruct((NC, N_PAD, HALF), jnp.float32),
        scratch_types=[
            pltpu.VMEM((NBLK, BLK), jnp.int32),
            pltpu.VMEM((NBLK, BLK), jnp.int32),
            pltpu.VMEM((NBUF, BLK, HALF), jnp.float32),
            pltpu.VMEM((2, HALF), jnp.float32),
            pltpu.VMEM_SHARED((N_PAD, HALF), jnp.float32),
        ]
        + [pltpu.SemaphoreType.DMA] * NBUF,
    )
    def k(t_r, src_r, dst_r, z_r, dinv_r, b_r, out_r,
          sidx, didx, rows_v, bv, acc, *sems):
        cid = lax.axis_index("c")
        sid = lax.axis_index("s")
        rows = pl.ds(sid * RPT, RPT)
        th = t_r.at[cid]

        def drain(bf):
            pltpu.make_async_copy(
                z_r.at[pl.ds(0, BLK)], rows_v.at[bf], sems[bf]
            ).wait()

        pltpu.sync_copy(z_r.at[rows], acc.at[rows])
        plsc.subcore_barrier()
        for m in range(NCHUNK):
            base_blk = sid * NBLKT + m * NBLK
            pltpu.sync_copy(src_r.at[pl.ds(base_blk, NBLK)], sidx)
            pltpu.sync_copy(dst_r.at[pl.ds(base_blk, NBLK)], didx)
            for bf in range(NBUF):
                pltpu.async_copy(th.at[sidx.at[bf]], rows_v.at[bf], sems[bf])

            def round_(g, carry):
                base = g * NBUF
                for bf in range(NBUF):
                    drain(bf)
                    pltpu.async_copy(
                        rows_v.at[bf], acc.at[didx.at[base + bf]], sems[bf],
                        add=True,
                    )
                for bf in range(NBUF):
                    nj = base + NBUF + bf

                    @pl.when(nj < NBLK)
                    def _(bf=bf, nj=nj):
                        drain(bf)
                        pltpu.async_copy(
                            th.at[sidx.at[nj]], rows_v.at[bf], sems[bf]
                        )

                return carry

            lax.fori_loop(0, NBLK // NBUF, round_, 0)
            for bf in range(NBUF):
                drain(bf)
        plsc.subcore_barrier()

        if not fuse_out:
            pltpu.sync_copy(acc.at[rows], out_r.at[cid].at[rows])
        else:
            pltpu.sync_copy(b_r.at[cid], bv.at[0])

            def chunk(c, carry):
                r0 = sid * RPT + c * CH
                pltpu.sync_copy(acc.at[pl.ds(r0, CH)],
                                rows_v.at[0, pl.ds(0, CH)])
                pltpu.sync_copy(th.at[pl.ds(r0, CH)],
                                rows_v.at[0, pl.ds(CH, CH)])
                pltpu.sync_copy(dinv_r.at[pl.ds(r0, CH)],
                                rows_v.at[1, pl.ds(0, CH)])

                def row(r, carry2):
                    for cc in range(HALF // 16):
                        cs = pl.ds(cc * 16, 16)
                        v = rows_v[0, r, cs] + rows_v[0, CH + r, cs]
                        rows_v[1, CH + r, cs] = (
                            v * rows_v[1, r, cs] + bv[0, cs]
                        )
                    return carry2

                lax.fori_loop(0, CH, row, 0)
                pltpu.sync_copy(
                    rows_v.at[1, pl.ds(CH, CH)],
                    out_r.at[cid].at[pl.ds(r0, CH)],
                )
                return carry

            lax.fori_loop(0, RPT // CH, chunk, 0)

    return k(t, src_blocks, dst_blocks, zeros128, dinv, b)


def _tc_stage1(x, W1, d0, d1):
    def body(x_r, w_r, d0_r, d1_r, t_r, dinv_r):
        deg = d0_r[:, :1] + d1_r[:, :1] + 1.0
        dinv = lax.rsqrt(deg)
        dinv_r[...] = jnp.broadcast_to(dinv, (RB, 16))
        t = jnp.dot(x_r[...] * dinv, w_r[...],
                    preferred_element_type=jnp.float32)
        t_r[0] = t[:, :HALF]
        t_r[1] = t[:, HALF:]

    return pl.pallas_call(
        body,
        grid=(N_PAD // RB,),
        in_specs=[
            pl.BlockSpec((RB, D), lambda i: (i, 0)),
            pl.BlockSpec((D, D), lambda i: (0, 0)),
            pl.BlockSpec((RB, 16), lambda i: (i, 0)),
            pl.BlockSpec((RB, 16), lambda i: (i, 0)),
        ],
        out_specs=[
            pl.BlockSpec((2, RB, HALF), lambda i: (0, i, 0)),
            pl.BlockSpec((RB, 16), lambda i: (i, 0)),
        ],
        out_shape=[
            jax.ShapeDtypeStruct((2, N_PAD, HALF), jnp.float32),
            jax.ShapeDtypeStruct((N_PAD, 16), jnp.float32),
        ],
    )(x, W1, d0, d1)


def _tc_stage2(agg, t1, dinv, W2, b1):
    def body(a_r, t_r, dinv_r, w_r, b_r, u_r, de_r):
        dinv = dinv_r[:, :1]
        b = b_r[...]
        w = w_r[...]
        z0 = jnp.maximum(
            (a_r[0] + t_r[0]) * dinv + b[:, :HALF], 0.0) * dinv
        z1 = jnp.maximum(
            (a_r[1] + t_r[1]) * dinv + b[:, HALF:], 0.0) * dinv
        h2 = (
            jnp.dot(z0, w[:HALF], preferred_element_type=jnp.float32)
            + jnp.dot(z1, w[HALF:], preferred_element_type=jnp.float32)
        )
        u_r[0] = h2[:, :HALF]
        u_r[1] = h2[:, HALF:]
        de_r[...] = jnp.broadcast_to(dinv, (RB, HALF))

    return pl.pallas_call(
        body,
        grid=(N_PAD // RB,),
        in_specs=[
            pl.BlockSpec((2, RB, HALF), lambda i: (0, i, 0)),
            pl.BlockSpec((2, RB, HALF), lambda i: (0, i, 0)),
            pl.BlockSpec((RB, 16), lambda i: (i, 0)),
            pl.BlockSpec((D, D), lambda i: (0, 0)),
            pl.BlockSpec((1, D), lambda i: (0, 0)),
        ],
        out_specs=[
            pl.BlockSpec((2, RB, HALF), lambda i: (0, i, 0)),
            pl.BlockSpec((RB, HALF), lambda i: (i, 0)),
        ],
        out_shape=[
            jax.ShapeDtypeStruct((2, N_PAD, HALF), jnp.float32),
            jax.ShapeDtypeStruct((N_PAD, HALF), jnp.float32),
        ],
    )(agg, t1, dinv, W2, b1)


def kernel(x, edge_index, W1, b1, W2, b2):
    src = edge_index[0].astype(jnp.int32)
    dst = edge_index[1].astype(jnp.int32)
    e = src.shape[0]
    padfill = jnp.full((E_PAD - e,), N_NODES, jnp.int32)
    srcb = jnp.concatenate([src, padfill]).reshape(NS * NBLKT, BLK)
    dstb = jnp.concatenate([dst, padfill]).reshape(NS * NBLKT, BLK)
    dstb_deg = dstb.reshape(NC * NS, NBLK, BLK)
    xp = jnp.zeros((N_PAD, D), jnp.float32).at[:N_NODES].set(x)
    z128 = jnp.zeros((N_PAD, HALF), jnp.float32)
    o128 = jnp.ones((BLK, HALF), jnp.float32)
    b2h = b2.reshape(2, HALF)

    degp = _sc_degree(dstb_deg, z128, o128)
    t1, dinv = _tc_stage1(xp, W1, degp[0, :, :16], degp[1, :, :16])
    agg = _sc_propagate(t1, srcb, dstb, z128, z128, b2h, fuse_out=False)
    t2, dinv_exp = _tc_stage2(agg, t1, dinv, W2, b1.reshape(1, D))
    outh = _sc_propagate(t2, srcb, dstb, z128, dinv_exp, b2h, fuse_out=True)
    out = jnp.concatenate([outh[0], outh[1]], axis=1)
    return out[:N_NODES]
